# Initial kernel scaffold; baseline (speedup 1.0000x reference)
import jax
import jax.numpy as jnp
from jax.experimental import pallas as pl


def kernel(x, edge, W1, b1, g1, bt1, W2, b2, g2, bt2, W3, b3):
    raise NotImplementedError("write your pallas kernel here")


# speedup vs baseline: 4.1767x; 4.1767x over previous
import functools

import jax
import jax.numpy as jnp
from jax import lax
from jax.experimental import pallas as pl
from jax.experimental.pallas import tpu as pltpu

N_HE = 10000
EPS = 1e-5
_INTERP = False


def _pick(b, n):
    return b if n % b == 0 else n


def _mm_plain(act, W):
    n, fin = act.shape
    fout = W.shape[1]
    rb = _pick(1000, n)

    def body(a_ref, w_ref, o_ref):
        o_ref[...] = jnp.dot(a_ref[...], w_ref[...],
                             preferred_element_type=jnp.float32).astype(jnp.bfloat16)

    return pl.pallas_call(
        body,
        grid=(n // rb,),
        in_specs=[pl.BlockSpec((rb, fin), lambda i: (i, 0)),
                  pl.BlockSpec((fin, fout), lambda i: (0, 0))],
        out_specs=pl.BlockSpec((rb, fout), lambda i: (i, 0)),
        out_shape=jax.ShapeDtypeStruct((n, fout), jnp.bfloat16),
        interpret=_INTERP,
    )(act, W)


def _mm_bn(h, stats, g2d, bt2d, W):
    n, fin = h.shape
    fout = W.shape[1]
    rb = _pick(1000, n)
    inv_n = 1.0 / n

    def body(h_ref, s_ref, g_ref, b_ref, w_ref, o_ref):
        mean = s_ref[0:1, :] * inv_n
        var = s_ref[1:2, :] * inv_n - mean * mean
        scale = g_ref[...] * lax.rsqrt(var + EPS)
        shift = b_ref[...] - mean * scale
        a = h_ref[...] * scale + shift
        o_ref[...] = jnp.dot(a, w_ref[...],
                             preferred_element_type=jnp.float32).astype(jnp.bfloat16)

    return pl.pallas_call(
        body,
        grid=(n // rb,),
        in_specs=[pl.BlockSpec((rb, fin), lambda i: (i, 0)),
                  pl.BlockSpec((2, fin), lambda i: (0, 0)),
                  pl.BlockSpec((1, fin), lambda i: (0, 0)),
                  pl.BlockSpec((1, fin), lambda i: (0, 0)),
                  pl.BlockSpec((fin, fout), lambda i: (0, 0))],
        out_specs=pl.BlockSpec((rb, fout), lambda i: (i, 0)),
        out_shape=jax.ShapeDtypeStruct((n, fout), jnp.bfloat16),
        interpret=_INTERP,
    )(h, stats, g2d, bt2d, W)


def _ht_mm(H, xw, binv_col):
    n, hep = H.shape
    f = xw.shape[1]
    cb = _pick(512, hep)

    def body(h_ref, x_ref, s_ref, o_ref):
        acc = lax.dot_general(h_ref[...], x_ref[...],
                              (((0,), (0,)), ((), ())),
                              preferred_element_type=jnp.float32)
        o_ref[...] = (acc * s_ref[...]).astype(jnp.bfloat16)

    return pl.pallas_call(
        body,
        grid=(hep // cb,),
        in_specs=[pl.BlockSpec((n, cb), lambda i: (0, i)),
                  pl.BlockSpec((n, f), lambda i: (0, 0)),
                  pl.BlockSpec((cb, 1), lambda i: (i, 0))],
        out_specs=pl.BlockSpec((cb, f), lambda i: (i, 0)),
        out_shape=jax.ShapeDtypeStruct((hep, f), jnp.bfloat16),
        interpret=_INTERP,
    )(H, xw, binv_col)


def _h_mm(H, t, dinv_col, b2d):
    n, hep = H.shape
    f = t.shape[1]
    rb = _pick(400, n)

    def body(h_ref, t_ref, s_ref, b_ref, o_ref):
        acc = jnp.dot(h_ref[...], t_ref[...], preferred_element_type=jnp.float32)
        o_ref[...] = jnp.maximum(acc * s_ref[...] + b_ref[...], 0.0)

    return pl.pallas_call(
        body,
        grid=(n // rb,),
        in_specs=[pl.BlockSpec((rb, hep), lambda i: (i, 0)),
                  pl.BlockSpec((hep, f), lambda i: (0, 0)),
                  pl.BlockSpec((rb, 1), lambda i: (i, 0)),
                  pl.BlockSpec((1, f), lambda i: (0, 0))],
        out_specs=pl.BlockSpec((rb, f), lambda i: (i, 0)),
        out_shape=jax.ShapeDtypeStruct((n, f), jnp.float32),
        interpret=_INTERP,
    )(H, t, dinv_col, b2d)


def _bn_stats(h):
    n, f = h.shape
    rb = _pick(1000, n)

    def body(h_ref, o_ref):
        i = pl.program_id(0)

        @pl.when(i == 0)
        def _():
            o_ref[...] = jnp.zeros_like(o_ref)

        v = h_ref[...]
        s = jnp.sum(v, axis=0, keepdims=True)
        ss = jnp.sum(v * v, axis=0, keepdims=True)
        o_ref[...] += jnp.concatenate([s, ss], axis=0)

    return pl.pallas_call(
        body,
        grid=(n // rb,),
        in_specs=[pl.BlockSpec((rb, f), lambda i: (i, 0))],
        out_specs=pl.BlockSpec((2, f), lambda i: (0, 0)),
        out_shape=jax.ShapeDtypeStruct((2, f), jnp.float32),
        interpret=_INTERP,
    )(h)


def _build_incidence(edge, n_nodes, n_he, hep):
    node, he = edge[0], edge[1]
    H = jnp.zeros((n_nodes, hep), jnp.bfloat16).at[node, he].add(jnp.bfloat16(1.0))
    D = jnp.zeros((n_nodes,), jnp.float32).at[node].add(1.0)
    B = jnp.zeros((hep,), jnp.float32).at[he].add(1.0)
    return H, D, B


def _encode(x, edge, W1, b1, g1, bt1, W2, b2, g2, bt2, W3, b3, n_he):
    n = x.shape[0]
    hep = ((n_he + 511) // 512) * 512
    H, D, B = _build_incidence(edge, n, n_he, hep)
    dinv = jnp.where(D > 0, 1.0 / D, 0.0).reshape(n, 1)
    binv = jnp.where(B > 0, 1.0 / B, 0.0).reshape(hep, 1)

    h = x
    for (W, b, g, bt) in ((W1, b1, None, None),
                          (W2, b2, g1, bt1),
                          (W3, b3, g2, bt2)):
        if g is None:
            xw = _mm_plain(h, W)
        else:
            stats = _bn_stats(h)
            xw = _mm_bn(h, stats, g.reshape(1, -1), bt.reshape(1, -1), W)
        t = _ht_mm(H, xw, binv)
        h = _h_mm(H, t, dinv, b.reshape(1, -1))
    return h


def kernel(x, edge, W1, b1, g1, bt1, W2, b2, g2, bt2, W3, b3):
    return _encode(x, edge, W1, b1, g1, bt1, W2, b2, g2, bt2, W3, b3, N_HE)
